# baseline (speedup 1.0000x reference)
import functools

import jax
import jax.numpy as jnp
from jax import lax
from jax.experimental import pallas as pl
from jax.experimental.pallas import tpu as pltpu
from jax.experimental.pallas import tpu_sc as plsc

N = 10000
E = 320000
F_IN = 128
H = 32
C = 2

NC = 2
NS = 16
L = 16
NW = NC * NS

CHUNK = 128
EPT = 10240
EP = NW * EPT
NCHUNKS = EPT // CHUNK
N_PAD = 10240
ROWS_PER_TILE = N_PAD // NS


NBUF = 4


def _scale_rows(rows_v, w_all, base):
    lane = lax.iota(jnp.int32, L)

    @plsc.parallel_loop(0, CHUNK // L)
    def _group(g):
        w16 = w_all[pl.ds(base + g * L, L)]
        rid = lane + g * L
        for j in range(H):
            cj = jnp.full((L,), j, jnp.int32)
            v = plsc.load_gather(rows_v, [rid, cj])
            plsc.store_scatter(rows_v, [rid, cj], v * w16)


def _conv_body(x0_hbm, src_hbm, dst_hbm, w_hbm, z_hbm, out_hbm,
               src_v, dst_v, w_all, rows_bufs, acc_sh, gsems, ssems):
    c = lax.axis_index("c")
    s = lax.axis_index("s")
    wid = c * NS + s

    pltpu.sync_copy(z_hbm.at[pl.ds(s * ROWS_PER_TILE, ROWS_PER_TILE)],
                    acc_sh.at[pl.ds(s * ROWS_PER_TILE, ROWS_PER_TILE)])
    pltpu.sync_copy(src_hbm.at[pl.ds(wid * NCHUNKS, NCHUNKS)], src_v)
    pltpu.sync_copy(dst_hbm.at[pl.ds(wid * NCHUNKS, NCHUNKS)], dst_v)
    pltpu.sync_copy(w_hbm.at[pl.ds(wid * EPT, EPT)], w_all)
    plsc.subcore_barrier()

    @pl.loop(0, NCHUNKS, step=NBUF)
    def _super(i0):
        gathers = []
        for b in range(NBUF):
            gathers.append(pltpu.async_copy(
                x0_hbm.at[src_v.at[i0 + b]], rows_bufs[b], gsems[b]))
        scatters = []
        for b in range(NBUF):
            gathers[b].wait()
            _scale_rows(rows_bufs[b], w_all, (i0 + b) * CHUNK)
            scatters.append(pltpu.async_copy(
                rows_bufs[b], acc_sh.at[dst_v.at[i0 + b]], ssems[b], add=True))
        for b in range(NBUF):
            scatters[b].wait()

    plsc.subcore_barrier()
    pltpu.sync_copy(acc_sh.at[pl.ds(s * ROWS_PER_TILE, ROWS_PER_TILE)],
                    out_hbm.at[c, pl.ds(s * ROWS_PER_TILE, ROWS_PER_TILE)])


@functools.cache
def _conv_sc_fn():
    return pl.kernel(
        _conv_body,
        out_type=jax.ShapeDtypeStruct((NC, N_PAD, H), jnp.float32),
        mesh=plsc.VectorSubcoreMesh(core_axis_name="c", subcore_axis_name="s",
                                    num_cores=NC, num_subcores=NS),
        compiler_params=pltpu.CompilerParams(use_tc_tiling_on_sc=False,
                                             needs_layout_passes=False),
        scratch_types=[
            pltpu.VMEM((NCHUNKS, CHUNK), jnp.int32),
            pltpu.VMEM((NCHUNKS, CHUNK), jnp.int32),
            pltpu.VMEM((EPT,), jnp.float32),
            [pltpu.VMEM((CHUNK, H), jnp.float32)] * NBUF,
            pltpu.VMEM_SHARED((N_PAD, H), jnp.float32),
            [pltpu.SemaphoreType.DMA] * NBUF,
            [pltpu.SemaphoreType.DMA] * NBUF,
        ],
    )


def _conv_sc(x0, src2d, dst2d, w_p, zeros):
    return _conv_sc_fn()(x0, src2d, dst2d, w_p, zeros)


def _first_lin_body(x_ref, w_ref, b_ref, o_ref):
    o_ref[...] = lax.dot_general(
        x_ref[...], w_ref[...], (((1,), (1,)), ((), ())),
        preferred_element_type=jnp.float32,
        precision=lax.Precision.HIGHEST) + b_ref[...]


def _gru_core(p_ref, h_ref, wir, wiz, win, whr, whz, whn,
              bir, biz, bin_, bhr, bhz, bhn):
    def dot(a, b):
        return lax.dot_general(a, b, (((1,), (1,)), ((), ())),
                               preferred_element_type=jnp.float32,
                               precision=lax.Precision.HIGHEST)
    xc = p_ref[0] + p_ref[1]
    h = h_ref[...]
    r = jax.nn.sigmoid(dot(xc, wir[...]) + bir[...] + dot(h, whr[...]) + bhr[...])
    z = jax.nn.sigmoid(dot(xc, wiz[...]) + biz[...] + dot(h, whz[...]) + bhz[...])
    n = jnp.tanh(dot(xc, win[...]) + bin_[...] + r * (dot(h, whn[...]) + bhn[...]))
    return (1.0 - z) * n + z * h


def _gru_lin1_body(p_ref, h_ref, wir, wiz, win, whr, whz, whn,
                   bir, biz, bin_, bhr, bhz, bhn, wl1, bl1, o_ref):
    x1 = _gru_core(p_ref, h_ref, wir, wiz, win, whr, whz, whn,
                   bir, biz, bin_, bhr, bhz, bhn)
    o_ref[...] = lax.dot_general(
        x1, wl1[...], (((1,), (1,)), ((), ())),
        preferred_element_type=jnp.float32,
        precision=lax.Precision.HIGHEST) + bl1[...]


def _gru_out_body(p_ref, h_ref, wir, wiz, win, whr, whz, whn,
                  bir, biz, bin_, bhr, bhz, bhn, wo, bo, o_ref):
    x2 = _gru_core(p_ref, h_ref, wir, wiz, win, whr, whz, whn,
                   bir, biz, bin_, bhr, bhz, bhn)
    logits = lax.dot_general(
        x2, wo[...], (((1,), (1,)), ((), ())),
        preferred_element_type=jnp.float32,
        precision=lax.Precision.HIGHEST) + bo[...]
    m = jnp.max(logits, axis=-1, keepdims=True)
    e = logits - m
    lse = jnp.log(jnp.sum(jnp.exp(e), axis=-1, keepdims=True))
    o_ref[...] = e - lse


BLK = 1280


def _full(shape):
    return pl.BlockSpec(shape, lambda i: (0,) * len(shape))


def kernel(x, edge_index, edge_weight, W_first, b_first, W_ih, W_hh,
           b_ih, b_hh, W_lin1, b_lin1, W_out, b_out):
    src = edge_index[0]
    dst = edge_index[1]
    pad = EP - E
    src2d = jnp.pad(src, (0, pad)).reshape(EP // CHUNK, CHUNK)
    dst2d = jnp.pad(dst, (0, pad)).reshape(EP // CHUNK, CHUNK)
    w_p = jnp.pad(edge_weight, (0, pad))
    zeros = jnp.zeros((N_PAD, H), jnp.float32)
    xp = jnp.pad(x, ((0, N_PAD - N), (0, 0)))

    wir, wiz, win = W_ih[:H], W_ih[H:2 * H], W_ih[2 * H:]
    whr, whz, whn = W_hh[:H], W_hh[H:2 * H], W_hh[2 * H:]
    bir, biz, bin_ = (b_ih[:H].reshape(1, H), b_ih[H:2 * H].reshape(1, H),
                      b_ih[2 * H:].reshape(1, H))
    bhr, bhz, bhn = (b_hh[:H].reshape(1, H), b_hh[H:2 * H].reshape(1, H),
                     b_hh[2 * H:].reshape(1, H))
    gru_w = (wir, wiz, win, whr, whz, whn, bir, biz, bin_, bhr, bhz, bhn)

    grid = (N_PAD // BLK,)
    row_blk = pl.BlockSpec((BLK, H), lambda i: (i, 0))
    p_blk = pl.BlockSpec((NC, BLK, H), lambda i: (0, i, 0))
    w32 = _full((H, H))
    b32 = _full((1, H))
    gru_specs = [w32] * 6 + [b32] * 6

    x0 = pl.pallas_call(
        _first_lin_body,
        grid=grid,
        in_specs=[pl.BlockSpec((BLK, F_IN), lambda i: (i, 0)),
                  _full((H, F_IN)), b32],
        out_specs=row_blk,
        out_shape=jax.ShapeDtypeStruct((N_PAD, H), jnp.float32),
    )(xp, W_first, b_first.reshape(1, H))

    p1 = _conv_sc(x0, src2d, dst2d, w_p, zeros)

    y = pl.pallas_call(
        _gru_lin1_body,
        grid=grid,
        in_specs=[p_blk, row_blk] + gru_specs + [w32, b32],
        out_specs=row_blk,
        out_shape=jax.ShapeDtypeStruct((N_PAD, H), jnp.float32),
    )(p1, x0, *gru_w, W_lin1, b_lin1.reshape(1, H))

    p2 = _conv_sc(y, src2d, dst2d, w_p, zeros)

    out = pl.pallas_call(
        _gru_out_body,
        grid=grid,
        in_specs=[p_blk, row_blk] + gru_specs + [_full((C, H)), _full((1, C))],
        out_specs=pl.BlockSpec((BLK, C), lambda i: (i, 0)),
        out_shape=jax.ShapeDtypeStruct((N_PAD, C), jnp.float32),
    )(p2, x0, *gru_w, W_out, b_out.reshape(1, C))

    return out[:N]


# speedup vs baseline: 1.8477x; 1.8477x over previous
import functools

import jax
import jax.numpy as jnp
from jax import lax
from jax.experimental import pallas as pl
from jax.experimental.pallas import tpu as pltpu
from jax.experimental.pallas import tpu_sc as plsc

N = 10000
E = 320000
F_IN = 128
H = 32
C = 2

NC = 2
NS = 16
L = 16
NW = NC * NS

CHUNK = 128
EPT = 10240
EP = NW * EPT
NCHUNKS = EPT // CHUNK
N_PAD = 10240
ROWS_PER_TILE = N_PAD // NS


NBUF = 4


def _scale_rows(rows_v, w_all, base):
    @plsc.parallel_loop(0, CHUNK // L)
    def _group(g):
        w16 = w_all[pl.ds(base + g * L, L)]
        for k in range(L):
            e = g * L + k
            wv = jnp.broadcast_to(w16[k], (L,))
            for half in range(H // L):
                sl = pl.ds(half * L, L)
                rows_v[e, sl] = rows_v[e, sl] * wv


def _conv_body(x0_hbm, src_hbm, dst_hbm, w_hbm, z_hbm, out_hbm,
               src_v, dst_v, w_all, rows_bufs, acc_sh, gsems, ssems):
    c = lax.axis_index("c")
    s = lax.axis_index("s")
    wid = c * NS + s

    pltpu.sync_copy(z_hbm.at[pl.ds(s * ROWS_PER_TILE, ROWS_PER_TILE)],
                    acc_sh.at[pl.ds(s * ROWS_PER_TILE, ROWS_PER_TILE)])
    pltpu.sync_copy(src_hbm.at[pl.ds(wid * NCHUNKS, NCHUNKS)], src_v)
    pltpu.sync_copy(dst_hbm.at[pl.ds(wid * NCHUNKS, NCHUNKS)], dst_v)
    pltpu.sync_copy(w_hbm.at[pl.ds(wid * EPT, EPT)], w_all)
    plsc.subcore_barrier()

    @pl.loop(0, NCHUNKS, step=NBUF)
    def _super(i0):
        gathers = []
        for b in range(NBUF):
            gathers.append(pltpu.async_copy(
                x0_hbm.at[src_v.at[i0 + b]], rows_bufs[b], gsems[b]))
        scatters = []
        for b in range(NBUF):
            gathers[b].wait()
            _scale_rows(rows_bufs[b], w_all, (i0 + b) * CHUNK)
            scatters.append(pltpu.async_copy(
                rows_bufs[b], acc_sh.at[dst_v.at[i0 + b]], ssems[b], add=True))
        for b in range(NBUF):
            scatters[b].wait()

    plsc.subcore_barrier()
    pltpu.sync_copy(acc_sh.at[pl.ds(s * ROWS_PER_TILE, ROWS_PER_TILE)],
                    out_hbm.at[c, pl.ds(s * ROWS_PER_TILE, ROWS_PER_TILE)])


@functools.cache
def _conv_sc_fn():
    return pl.kernel(
        _conv_body,
        out_type=jax.ShapeDtypeStruct((NC, N_PAD, H), jnp.float32),
        mesh=plsc.VectorSubcoreMesh(core_axis_name="c", subcore_axis_name="s",
                                    num_cores=NC, num_subcores=NS),
        compiler_params=pltpu.CompilerParams(use_tc_tiling_on_sc=False,
                                             needs_layout_passes=False),
        scratch_types=[
            pltpu.VMEM((NCHUNKS, CHUNK), jnp.int32),
            pltpu.VMEM((NCHUNKS, CHUNK), jnp.int32),
            pltpu.VMEM((EPT,), jnp.float32),
            [pltpu.VMEM((CHUNK, H), jnp.float32)] * NBUF,
            pltpu.VMEM_SHARED((N_PAD, H), jnp.float32),
            [pltpu.SemaphoreType.DMA] * NBUF,
            [pltpu.SemaphoreType.DMA] * NBUF,
        ],
    )


def _conv_sc(x0, src2d, dst2d, w_p, zeros):
    return _conv_sc_fn()(x0, src2d, dst2d, w_p, zeros)


def _first_lin_body(x_ref, w_ref, b_ref, o_ref):
    o_ref[...] = lax.dot_general(
        x_ref[...], w_ref[...], (((1,), (1,)), ((), ())),
        preferred_element_type=jnp.float32,
        precision=lax.Precision.HIGHEST) + b_ref[...]


def _gru_core(p_ref, h_ref, wir, wiz, win, whr, whz, whn,
              bir, biz, bin_, bhr, bhz, bhn):
    def dot(a, b):
        return lax.dot_general(a, b, (((1,), (1,)), ((), ())),
                               preferred_element_type=jnp.float32,
                               precision=lax.Precision.HIGHEST)
    xc = p_ref[0] + p_ref[1]
    h = h_ref[...]
    r = jax.nn.sigmoid(dot(xc, wir[...]) + bir[...] + dot(h, whr[...]) + bhr[...])
    z = jax.nn.sigmoid(dot(xc, wiz[...]) + biz[...] + dot(h, whz[...]) + bhz[...])
    n = jnp.tanh(dot(xc, win[...]) + bin_[...] + r * (dot(h, whn[...]) + bhn[...]))
    return (1.0 - z) * n + z * h


def _gru_lin1_body(p_ref, h_ref, wir, wiz, win, whr, whz, whn,
                   bir, biz, bin_, bhr, bhz, bhn, wl1, bl1, o_ref):
    x1 = _gru_core(p_ref, h_ref, wir, wiz, win, whr, whz, whn,
                   bir, biz, bin_, bhr, bhz, bhn)
    o_ref[...] = lax.dot_general(
        x1, wl1[...], (((1,), (1,)), ((), ())),
        preferred_element_type=jnp.float32,
        precision=lax.Precision.HIGHEST) + bl1[...]


def _gru_out_body(p_ref, h_ref, wir, wiz, win, whr, whz, whn,
                  bir, biz, bin_, bhr, bhz, bhn, wo, bo, o_ref):
    x2 = _gru_core(p_ref, h_ref, wir, wiz, win, whr, whz, whn,
                   bir, biz, bin_, bhr, bhz, bhn)
    logits = lax.dot_general(
        x2, wo[...], (((1,), (1,)), ((), ())),
        preferred_element_type=jnp.float32,
        precision=lax.Precision.HIGHEST) + bo[...]
    m = jnp.max(logits, axis=-1, keepdims=True)
    e = logits - m
    lse = jnp.log(jnp.sum(jnp.exp(e), axis=-1, keepdims=True))
    o_ref[...] = e - lse


BLK = 1280


def _full(shape):
    return pl.BlockSpec(shape, lambda i: (0,) * len(shape))


def kernel(x, edge_index, edge_weight, W_first, b_first, W_ih, W_hh,
           b_ih, b_hh, W_lin1, b_lin1, W_out, b_out):
    src = edge_index[0]
    dst = edge_index[1]
    pad = EP - E
    src2d = jnp.pad(src, (0, pad)).reshape(EP // CHUNK, CHUNK)
    dst2d = jnp.pad(dst, (0, pad)).reshape(EP // CHUNK, CHUNK)
    w_p = jnp.pad(edge_weight, (0, pad))
    zeros = jnp.zeros((N_PAD, H), jnp.float32)
    xp = jnp.pad(x, ((0, N_PAD - N), (0, 0)))

    wir, wiz, win = W_ih[:H], W_ih[H:2 * H], W_ih[2 * H:]
    whr, whz, whn = W_hh[:H], W_hh[H:2 * H], W_hh[2 * H:]
    bir, biz, bin_ = (b_ih[:H].reshape(1, H), b_ih[H:2 * H].reshape(1, H),
                      b_ih[2 * H:].reshape(1, H))
    bhr, bhz, bhn = (b_hh[:H].reshape(1, H), b_hh[H:2 * H].reshape(1, H),
                     b_hh[2 * H:].reshape(1, H))
    gru_w = (wir, wiz, win, whr, whz, whn, bir, biz, bin_, bhr, bhz, bhn)

    grid = (N_PAD // BLK,)
    row_blk = pl.BlockSpec((BLK, H), lambda i: (i, 0))
    p_blk = pl.BlockSpec((NC, BLK, H), lambda i: (0, i, 0))
    w32 = _full((H, H))
    b32 = _full((1, H))
    gru_specs = [w32] * 6 + [b32] * 6

    x0 = pl.pallas_call(
        _first_lin_body,
        grid=grid,
        in_specs=[pl.BlockSpec((BLK, F_IN), lambda i: (i, 0)),
                  _full((H, F_IN)), b32],
        out_specs=row_blk,
        out_shape=jax.ShapeDtypeStruct((N_PAD, H), jnp.float32),
    )(xp, W_first, b_first.reshape(1, H))

    p1 = _conv_sc(x0, src2d, dst2d, w_p, zeros)

    y = pl.pallas_call(
        _gru_lin1_body,
        grid=grid,
        in_specs=[p_blk, row_blk] + gru_specs + [w32, b32],
        out_specs=row_blk,
        out_shape=jax.ShapeDtypeStruct((N_PAD, H), jnp.float32),
    )(p1, x0, *gru_w, W_lin1, b_lin1.reshape(1, H))

    p2 = _conv_sc(y, src2d, dst2d, w_p, zeros)

    out = pl.pallas_call(
        _gru_out_body,
        grid=grid,
        in_specs=[p_blk, row_blk] + gru_specs + [_full((C, H)), _full((1, C))],
        out_specs=pl.BlockSpec((BLK, C), lambda i: (i, 0)),
        out_shape=jax.ShapeDtypeStruct((N_PAD, C), jnp.float32),
    )(p2, x0, *gru_w, W_out, b_out.reshape(1, C))

    return out[:N]


# speedup vs baseline: 3.2089x; 1.7367x over previous
import functools

import jax
import jax.numpy as jnp
from jax import lax
from jax.experimental import pallas as pl
from jax.experimental.pallas import tpu as pltpu
from jax.experimental.pallas import tpu_sc as plsc

N = 10000
E = 320000
F_IN = 128
H = 32
C = 2

NC = 2
NS = 16
L = 16
NW = NC * NS

CHUNK = 128
EPT = 10240
EP = NW * EPT
NCHUNKS = EPT // CHUNK
N_PAD = 10240
ROWS_PER_TILE = N_PAD // NS


NBUF = 4


def _scale_rows(rows_v, w_all, base):
    @plsc.parallel_loop(0, CHUNK // L)
    def _group(g):
        w16 = w_all[pl.ds(base + g * L, L)]
        for k in range(L):
            e = g * L + k
            wv = jnp.broadcast_to(w16[k], (L,))
            for half in range(H // L):
                sl = pl.ds(half * L, L)
                rows_v[e, sl] = rows_v[e, sl] * wv


def _conv_body(x0_hbm, src_hbm, dst_hbm, w_hbm, z_hbm, out_hbm,
               src_v, dst_v, w_all, rows_bufs, acc_sh, x0_sh, gsems, ssems):
    c = lax.axis_index("c")
    s = lax.axis_index("s")
    wid = c * NS + s
    stripe = pl.ds(s * ROWS_PER_TILE, ROWS_PER_TILE)

    pltpu.sync_copy(x0_hbm.at[stripe], x0_sh.at[stripe])
    pltpu.sync_copy(z_hbm.at[stripe], acc_sh.at[stripe])
    pltpu.sync_copy(src_hbm.at[pl.ds(wid * NCHUNKS, NCHUNKS)], src_v)
    pltpu.sync_copy(dst_hbm.at[pl.ds(wid * NCHUNKS, NCHUNKS)], dst_v)
    pltpu.sync_copy(w_hbm.at[pl.ds(wid * EPT, EPT)], w_all)
    plsc.subcore_barrier()

    @pl.loop(0, NCHUNKS, step=NBUF)
    def _super(i0):
        gathers = []
        for b in range(NBUF):
            gathers.append(pltpu.async_copy(
                x0_sh.at[src_v.at[i0 + b]], rows_bufs[b], gsems[b]))
        scatters = []
        for b in range(NBUF):
            gathers[b].wait()
            _scale_rows(rows_bufs[b], w_all, (i0 + b) * CHUNK)
            scatters.append(pltpu.async_copy(
                rows_bufs[b], acc_sh.at[dst_v.at[i0 + b]], ssems[b], add=True))
        for b in range(NBUF):
            scatters[b].wait()

    plsc.subcore_barrier()
    pltpu.sync_copy(acc_sh.at[pl.ds(s * ROWS_PER_TILE, ROWS_PER_TILE)],
                    out_hbm.at[c, pl.ds(s * ROWS_PER_TILE, ROWS_PER_TILE)])


@functools.cache
def _conv_sc_fn():
    return pl.kernel(
        _conv_body,
        out_type=jax.ShapeDtypeStruct((NC, N_PAD, H), jnp.float32),
        mesh=plsc.VectorSubcoreMesh(core_axis_name="c", subcore_axis_name="s",
                                    num_cores=NC, num_subcores=NS),
        compiler_params=pltpu.CompilerParams(use_tc_tiling_on_sc=False,
                                             needs_layout_passes=False),
        scratch_types=[
            pltpu.VMEM((NCHUNKS, CHUNK), jnp.int32),
            pltpu.VMEM((NCHUNKS, CHUNK), jnp.int32),
            pltpu.VMEM((EPT,), jnp.float32),
            [pltpu.VMEM((CHUNK, H), jnp.float32)] * NBUF,
            pltpu.VMEM_SHARED((N_PAD, H), jnp.float32),
            pltpu.VMEM_SHARED((N_PAD, H), jnp.float32),
            [pltpu.SemaphoreType.DMA] * NBUF,
            [pltpu.SemaphoreType.DMA] * NBUF,
        ],
    )


def _conv_sc(x0, src2d, dst2d, w_p, zeros):
    return _conv_sc_fn()(x0, src2d, dst2d, w_p, zeros)


def _first_lin_body(x_ref, w_ref, b_ref, o_ref):
    o_ref[...] = lax.dot_general(
        x_ref[...], w_ref[...], (((1,), (1,)), ((), ())),
        preferred_element_type=jnp.float32,
        precision=lax.Precision.HIGHEST) + b_ref[...]


def _gru_core(p_ref, h_ref, wir, wiz, win, whr, whz, whn,
              bir, biz, bin_, bhr, bhz, bhn):
    def dot(a, b):
        return lax.dot_general(a, b, (((1,), (1,)), ((), ())),
                               preferred_element_type=jnp.float32,
                               precision=lax.Precision.HIGHEST)
    xc = p_ref[0] + p_ref[1]
    h = h_ref[...]
    r = jax.nn.sigmoid(dot(xc, wir[...]) + bir[...] + dot(h, whr[...]) + bhr[...])
    z = jax.nn.sigmoid(dot(xc, wiz[...]) + biz[...] + dot(h, whz[...]) + bhz[...])
    n = jnp.tanh(dot(xc, win[...]) + bin_[...] + r * (dot(h, whn[...]) + bhn[...]))
    return (1.0 - z) * n + z * h


def _gru_lin1_body(p_ref, h_ref, wir, wiz, win, whr, whz, whn,
                   bir, biz, bin_, bhr, bhz, bhn, wl1, bl1, o_ref):
    x1 = _gru_core(p_ref, h_ref, wir, wiz, win, whr, whz, whn,
                   bir, biz, bin_, bhr, bhz, bhn)
    o_ref[...] = lax.dot_general(
        x1, wl1[...], (((1,), (1,)), ((), ())),
        preferred_element_type=jnp.float32,
        precision=lax.Precision.HIGHEST) + bl1[...]


def _gru_out_body(p_ref, h_ref, wir, wiz, win, whr, whz, whn,
                  bir, biz, bin_, bhr, bhz, bhn, wo, bo, o_ref):
    x2 = _gru_core(p_ref, h_ref, wir, wiz, win, whr, whz, whn,
                   bir, biz, bin_, bhr, bhz, bhn)
    logits = lax.dot_general(
        x2, wo[...], (((1,), (1,)), ((), ())),
        preferred_element_type=jnp.float32,
        precision=lax.Precision.HIGHEST) + bo[...]
    m = jnp.max(logits, axis=-1, keepdims=True)
    e = logits - m
    lse = jnp.log(jnp.sum(jnp.exp(e), axis=-1, keepdims=True))
    o_ref[...] = e - lse


BLK = 1280


def _full(shape):
    return pl.BlockSpec(shape, lambda i: (0,) * len(shape))


def kernel(x, edge_index, edge_weight, W_first, b_first, W_ih, W_hh,
           b_ih, b_hh, W_lin1, b_lin1, W_out, b_out):
    src = edge_index[0]
    dst = edge_index[1]
    pad = EP - E
    src2d = jnp.pad(src, (0, pad)).reshape(EP // CHUNK, CHUNK)
    dst2d = jnp.pad(dst, (0, pad)).reshape(EP // CHUNK, CHUNK)
    w_p = jnp.pad(edge_weight, (0, pad))
    zeros = jnp.zeros((N_PAD, H), jnp.float32)
    xp = jnp.pad(x, ((0, N_PAD - N), (0, 0)))

    wir, wiz, win = W_ih[:H], W_ih[H:2 * H], W_ih[2 * H:]
    whr, whz, whn = W_hh[:H], W_hh[H:2 * H], W_hh[2 * H:]
    bir, biz, bin_ = (b_ih[:H].reshape(1, H), b_ih[H:2 * H].reshape(1, H),
                      b_ih[2 * H:].reshape(1, H))
    bhr, bhz, bhn = (b_hh[:H].reshape(1, H), b_hh[H:2 * H].reshape(1, H),
                     b_hh[2 * H:].reshape(1, H))
    gru_w = (wir, wiz, win, whr, whz, whn, bir, biz, bin_, bhr, bhz, bhn)

    grid = (N_PAD // BLK,)
    row_blk = pl.BlockSpec((BLK, H), lambda i: (i, 0))
    p_blk = pl.BlockSpec((NC, BLK, H), lambda i: (0, i, 0))
    w32 = _full((H, H))
    b32 = _full((1, H))
    gru_specs = [w32] * 6 + [b32] * 6

    x0 = pl.pallas_call(
        _first_lin_body,
        grid=grid,
        in_specs=[pl.BlockSpec((BLK, F_IN), lambda i: (i, 0)),
                  _full((H, F_IN)), b32],
        out_specs=row_blk,
        out_shape=jax.ShapeDtypeStruct((N_PAD, H), jnp.float32),
    )(xp, W_first, b_first.reshape(1, H))

    p1 = _conv_sc(x0, src2d, dst2d, w_p, zeros)

    y = pl.pallas_call(
        _gru_lin1_body,
        grid=grid,
        in_specs=[p_blk, row_blk] + gru_specs + [w32, b32],
        out_specs=row_blk,
        out_shape=jax.ShapeDtypeStruct((N_PAD, H), jnp.float32),
    )(p1, x0, *gru_w, W_lin1, b_lin1.reshape(1, H))

    p2 = _conv_sc(y, src2d, dst2d, w_p, zeros)

    out = pl.pallas_call(
        _gru_out_body,
        grid=grid,
        in_specs=[p_blk, row_blk] + gru_specs + [_full((C, H)), _full((1, C))],
        out_specs=pl.BlockSpec((BLK, C), lambda i: (i, 0)),
        out_shape=jax.ShapeDtypeStruct((N_PAD, C), jnp.float32),
    )(p2, x0, *gru_w, W_out, b_out.reshape(1, C))

    return out[:N]
